# baseline (speedup 1.0000x reference)
import functools

import jax
import jax.numpy as jnp
import numpy as np
from jax.experimental import pallas as pl
from jax.experimental.pallas import tpu as pltpu
from jax.sharding import Mesh, PartitionSpec as P

_N = 256
_NM = 248
_PAD = _N - _NM
_ITERS = 3
_LR = 0.1
_B1, _B2, _EPS = 0.9, 0.999, 1e-8
_UB = 8

_INV_2NM = 1.0 / (2.0 * _NM)
_INV_MEAN = 1.0 / (_NM * _N)


def _lane_prod(x):
    w = x.shape[-1]
    while w > 1:
        w //= 2
        x = x[..., :w] * x[..., w:]
    return x


def _masked_sigmoid(hlat):
    lane = jax.lax.broadcasted_iota(jnp.int32, hlat.shape, hlat.ndim - 1)
    return jnp.where(lane >= _PAD, jax.nn.sigmoid(hlat), 0.0)


def _grad_odd_body(rows, didx_ref, aw2_ref, hlat_ref, gh_ref):
    u0 = didx_ref[0] * rows + pl.program_id(0) * _UB
    aw2 = aw2_ref[...]
    h = _masked_sigmoid(hlat_ref[...])
    f = 1.0 - aw2[None, :, :] * h[:, None, :]
    fz = f == 0.0
    zc = jnp.sum(jnp.where(fz, 1.0, 0.0), axis=2, keepdims=True)
    praw = _lane_prod(f)
    w_io = jax.lax.broadcasted_iota(jnp.int32, (_UB, _NM, 1), 1)
    u_io = jax.lax.broadcasted_iota(jnp.int32, (_UB, _NM, 1), 0)
    tcol = jnp.where(w_io == u_io + u0, -1.0, 1.0)
    gprod = (praw - tcol) * _INV_2NM

    def fast(_):
        contrib = ((gprod * praw) / f) * aw2[None, :, :]
        return -jnp.sum(contrib, axis=1)

    def slow(_):
        f_safe = jnp.where(fz, 1.0, f)
        nzprod = _lane_prod(f_safe)
        q = jnp.where(fz, jnp.where(zc == 1.0, 1.0, 0.0),
                      jnp.where(zc == 0.0, 1.0 / f_safe, 0.0))
        contrib = ((gprod * nzprod) * q) * aw2[None, :, :]
        return -jnp.sum(contrib, axis=1)

    gh_ref[...] = jax.lax.cond(jnp.max(zc) > 0.0, slow, fast, 0)


def _tau_adam(tau, mt, vt, gtau, c1p, c2p):
    mt_n = _B1 * mt + (1.0 - _B1) * gtau
    vt_n = _B2 * vt + (1.0 - _B2) * (gtau * gtau)
    tau_n = tau - _LR * (mt_n / c1p) / (jnp.sqrt(vt_n / c2p) + _EPS)
    return tau_n, mt_n, vt_n


def _update_body(rows, cc_ref, didx_ref, hlat_ref, tau_ref, mg_ref, vg_ref,
                 mt_ref, vt_ref, gh_ref, st_ref,
                 hlat_o, tau_o, mg_o, vg_o, mt_o, vt_o, sp_o):
    c1p, c2p, c1g, c2g = (cc_ref[0], cc_ref[1], cc_ref[2], cc_ref[3])
    d0 = didx_ref[0] * rows

    tau, mt_n, vt_n = _tau_adam(tau_ref[...], mt_ref[...], vt_ref[...],
                                st_ref[...] * _INV_MEAN, c1p, c2p)
    tau_o[...] = tau
    mt_o[...] = mt_n
    vt_o[...] = vt_n

    hlat = hlat_ref[...]
    h = _masked_sigmoid(hlat)
    tau_colF = jnp.transpose(tau)
    if rows == _N:
        tau_col = tau_colF
    else:
        tau_col = jnp.where(didx_ref[0] == 0,
                            tau_colF[:rows, :], tau_colF[_N - rows:, :])
    r = jnp.maximum(tau_col - tau + 0.1, 0.0)
    gh = gh_ref[...] + (r * r) * _INV_MEAN
    ghlat = gh * (h * (1.0 - h))

    row_g = d0 + jax.lax.broadcasted_iota(jnp.int32, (rows, _N), 0)
    w_mat = jnp.where(row_g < _NM, h * (2.0 * r), 0.0)
    colsum = jnp.sum(w_mat, axis=0, keepdims=True)
    rowsum = jnp.sum(w_mat, axis=1, keepdims=True)
    rs_row = jnp.transpose(rowsum)
    if rows == _N:
        rs_embed = rs_row
    else:
        z = jnp.zeros((1, _N - rows), jnp.float32)
        rs_embed = jnp.where(didx_ref[0] == 0,
                             jnp.concatenate([rs_row, z], axis=1),
                             jnp.concatenate([z, rs_row], axis=1))
    sp_o[...] = rs_embed - colsum

    mg = _B1 * mg_ref[...] + (1.0 - _B1) * ghlat
    vg = _B2 * vg_ref[...] + (1.0 - _B2) * (ghlat * ghlat)
    mg_o[...] = mg
    vg_o[...] = vg
    hlat_o[...] = hlat - _LR * (mg / c1g) / (jnp.sqrt(vg / c2g) + _EPS)


def _loss_body(rows, nblk, cc_ref, didx_ref, aw2_ref, hlat_ref, tau_ref,
               mt_ref, vt_ref, st_ref, out_ref, acc_ref, tr_ref, tc_ref):
    i = pl.program_id(0)
    d0 = didx_ref[0] * rows
    u0 = d0 + i * _UB

    @pl.when(i == 0)
    def _():
        tau4, _, _ = _tau_adam(tau_ref[...], mt_ref[...], vt_ref[...],
                               st_ref[...] * _INV_MEAN, cc_ref[0], cc_ref[1])
        tr_ref[...] = tau4
        tc_ref[...] = jnp.transpose(tau4)

    tau = tr_ref[...]
    aw2 = aw2_ref[...]
    h = _masked_sigmoid(hlat_ref[...])
    f = 1.0 - aw2[None, :, :] * h[:, None, :]
    praw = _lane_prod(f)
    w_io = jax.lax.broadcasted_iota(jnp.int32, (_UB, _NM, 1), 1)
    u_io = jax.lax.broadcasted_iota(jnp.int32, (_UB, _NM, 1), 0)
    tcol = jnp.where(w_io == u_io + u0, -1.0, 1.0)
    umask = (u_io + u0) < _NM
    se = jnp.where(umask, (praw - tcol) ** 2, 0.0)
    odd = jnp.sum(se)

    tau_col = tc_ref[pl.ds(u0, _UB), :]
    r = jnp.maximum(tau_col - tau + 0.1, 0.0)
    rmask = (u0 + jax.lax.broadcasted_iota(jnp.int32, (_UB, _N), 0)) < _NM
    order = jnp.sum(jnp.where(rmask, h * (r * r), 0.0))

    part = odd / (4.0 * _NM) + order * _INV_MEAN

    @pl.when(i == 0)
    def _():
        acc_ref[0] = 0.0

    acc_ref[0] += part

    @pl.when(i == nblk - 1)
    def _():
        out_ref[0, 0] = acc_ref[0]


@functools.lru_cache(maxsize=None)
def _build_calls(rows):
    nblk = rows // _UB
    f32 = jnp.float32
    sd = jax.ShapeDtypeStruct

    grad_call = pl.pallas_call(
        functools.partial(_grad_odd_body, rows),
        grid=(nblk,),
        in_specs=[
            pl.BlockSpec(memory_space=pltpu.SMEM),
            pl.BlockSpec((_NM, _N), lambda i: (0, 0)),
            pl.BlockSpec((_UB, _N), lambda i: (i, 0)),
        ],
        out_specs=pl.BlockSpec((_UB, _N), lambda i: (i, 0)),
        out_shape=sd((rows, _N), f32),
    )

    big = pl.BlockSpec((rows, _N), lambda: (0, 0))
    row = pl.BlockSpec((1, _N), lambda: (0, 0))
    update_call = pl.pallas_call(
        functools.partial(_update_body, rows),
        in_specs=[pl.BlockSpec(memory_space=pltpu.SMEM),
                  pl.BlockSpec(memory_space=pltpu.SMEM),
                  big, row, big, big, row, row, big, row],
        out_specs=[big, row, big, big, row, row, row],
        out_shape=[sd((rows, _N), f32), sd((1, _N), f32),
                   sd((rows, _N), f32), sd((rows, _N), f32),
                   sd((1, _N), f32), sd((1, _N), f32), sd((1, _N), f32)],
        input_output_aliases={2: 0, 3: 1, 4: 2, 5: 3, 6: 4, 7: 5},
    )

    loss_call = pl.pallas_call(
        functools.partial(_loss_body, rows, nblk),
        grid=(nblk,),
        in_specs=[
            pl.BlockSpec(memory_space=pltpu.SMEM),
            pl.BlockSpec(memory_space=pltpu.SMEM),
            pl.BlockSpec((_NM, _N), lambda i: (0, 0)),
            pl.BlockSpec((_UB, _N), lambda i: (i, 0)),
            pl.BlockSpec((1, _N), lambda i: (0, 0)),
            pl.BlockSpec((1, _N), lambda i: (0, 0)),
            pl.BlockSpec((1, _N), lambda i: (0, 0)),
            pl.BlockSpec((1, _N), lambda i: (0, 0)),
        ],
        out_specs=pl.BlockSpec(memory_space=pltpu.SMEM),
        out_shape=sd((1, 1), f32),
        scratch_shapes=[pltpu.SMEM((1,), f32),
                        pltpu.VMEM((1, _N), f32),
                        pltpu.VMEM((_N, 1), f32)],
    )

    return grad_call, update_call, loss_call


def kernel(A, tau_init, G_latent_init):
    devs = jax.devices()
    nd = 2 if len(devs) >= 2 else 1
    rows = _N // nd
    mesh = Mesh(np.array(devs[:nd]), ("d",))
    grad_call, update_call, loss_call = _build_calls(rows)

    aw2 = 2.0 * A[: _NM, :]
    hlat0 = jnp.pad(G_latent_init, ((0, _PAD), (_PAD, 0)))
    tau0 = tau_init.reshape(1, _N)

    def _cc(t):
        c = lambda b, s: 1.0 - b ** s if s >= 1 else 1.0
        return jnp.asarray(np.array(
            [c(_B1, t - 1), c(_B2, t - 1), c(_B1, t), c(_B2, t)],
            dtype=np.float32))

    def run(aw2, hlat, tau):
        didx = jax.lax.axis_index("d").astype(jnp.int32).reshape(1)
        z_big = jnp.zeros((rows, _N), jnp.float32)
        z_row = jnp.zeros((1, _N), jnp.float32)
        mg, vg, mt, vt, stats = z_big, z_big, z_row, z_row, z_row
        for t in range(1, _ITERS + 1):
            gh = grad_call(didx, aw2, hlat)
            hlat, tau, mg, vg, mt, vt, sp = update_call(
                _cc(t), didx, hlat, tau, mg, vg, mt, vt, gh, stats)
            stats = jax.lax.psum(sp, "d")
        part = loss_call(_cc(_ITERS + 1), didx, aw2, hlat, tau, mt, vt,
                         stats)
        return jax.lax.psum(part[0, 0], "d")

    shd = jax.shard_map(run, mesh=mesh, check_vma=False,
                        in_specs=(P(None, None), P("d", None), P(None, None)),
                        out_specs=P())
    return shd(aw2, hlat0, tau0)


# speedup vs baseline: 2.1779x; 2.1779x over previous
import jax
import jax.numpy as jnp
from jax.experimental import pallas as pl
from jax.experimental.pallas import tpu as pltpu

_N = 256
_NM = 248
_PAD = _N - _NM
_ITERS = 3
_LR = 0.1
_B1, _B2, _EPS = 0.9, 0.999, 1e-8
_UB = 8
_NBLK = _NM // _UB

_INV_2NM = 1.0 / (2.0 * _NM)
_INV_MEAN = 1.0 / (_NM * _N)


def _lane_prod(x):
    w = x.shape[-1]
    while w > 1:
        w //= 2
        x = x[..., :w] * x[..., w:]
    return x


def _body(aw2_ref, hlat0_ref, tau0_ref, out_ref,
          hlat_ref, h_ref, gh_ref, mg_ref, vg_ref,
          tau_ref, mt_ref, vt_ref):
    aw2 = aw2_ref[...]

    lane = jax.lax.broadcasted_iota(jnp.int32, (_NM, _N), 1)
    colmask = lane >= _PAD

    hlat_ref[...] = hlat0_ref[...]
    tau_ref[...] = tau0_ref[...]
    mg_ref[...] = jnp.zeros((_NM, _N), jnp.float32)
    vg_ref[...] = jnp.zeros((_NM, _N), jnp.float32)
    mt_ref[...] = jnp.zeros((1, _N), jnp.float32)
    vt_ref[...] = jnp.zeros((1, _N), jnp.float32)

    w_io = jax.lax.broadcasted_iota(jnp.int32, (_UB, _NM, 1), 1)
    u_io = jax.lax.broadcasted_iota(jnp.int32, (_UB, _NM, 1), 0)

    def block_fwd(u0):
        h_blk = h_ref[pl.ds(u0, _UB), :]
        f = 1.0 - aw2[None, :, :] * h_blk[:, None, :]
        praw = _lane_prod(f)
        tcol = jnp.where(w_io == u_io + u0, -1.0, 1.0)
        return f, praw, tcol

    def grad_step(t):
        hlat = hlat_ref[...]
        h = jnp.where(colmask, jax.nn.sigmoid(hlat), 0.0)
        h_ref[...] = h

        def blk(i, carry):
            u0 = i * _UB
            f, praw, tcol = block_fwd(u0)
            gprod = (praw - tcol) * _INV_2NM

            def fast(_):
                contrib = ((gprod * praw) / f) * aw2[None, :, :]
                return -jnp.sum(contrib, axis=1)

            def slow(_):
                fz = f == 0.0
                zc = jnp.sum(jnp.where(fz, 1.0, 0.0), axis=2, keepdims=True)
                f_safe = jnp.where(fz, 1.0, f)
                nzprod = _lane_prod(f_safe)
                q = jnp.where(fz, jnp.where(zc == 1.0, 1.0, 0.0),
                              jnp.where(zc == 0.0, 1.0 / f_safe, 0.0))
                contrib = ((gprod * nzprod) * q) * aw2[None, :, :]
                return -jnp.sum(contrib, axis=1)

            has_zero = jnp.min(jnp.abs(f)) == 0.0
            gh_ref[pl.ds(u0, _UB), :] = jax.lax.cond(has_zero, slow, fast, 0)
            return carry

        jax.lax.fori_loop(0, _NBLK, blk, 0, unroll=False)

        tau = tau_ref[...]
        tau_col = jnp.transpose(tau)[: _NM, :]
        d = tau_col - tau + 0.1
        r = jnp.maximum(d, 0.0)
        h = h_ref[...]
        gh = gh_ref[...] + (r * r) * _INV_MEAN
        ghlat = gh * (h * (1.0 - h))

        w_mat = h * (2.0 * r)
        rowsum = jnp.sum(w_mat, axis=1, keepdims=True)
        rowsum_full = jnp.concatenate(
            [rowsum, jnp.zeros((_PAD, 1), jnp.float32)], axis=0)
        gtau = (jnp.transpose(rowsum_full)
                - jnp.sum(w_mat, axis=0, keepdims=True)) * _INV_MEAN

        c1 = 1.0 - _B1 ** t
        c2 = 1.0 - _B2 ** t
        mg = _B1 * mg_ref[...] + (1.0 - _B1) * ghlat
        vg = _B2 * vg_ref[...] + (1.0 - _B2) * (ghlat * ghlat)
        mg_ref[...] = mg
        vg_ref[...] = vg
        hlat_ref[...] = hlat - _LR * (mg / c1) / (jnp.sqrt(vg / c2) + _EPS)

        mt = _B1 * mt_ref[...] + (1.0 - _B1) * gtau
        vt = _B2 * vt_ref[...] + (1.0 - _B2) * (gtau * gtau)
        mt_ref[...] = mt
        vt_ref[...] = vt
        tau_ref[...] = tau - _LR * (mt / c1) / (jnp.sqrt(vt / c2) + _EPS)

    for t in range(1, _ITERS + 1):
        grad_step(t)

    hlat = hlat_ref[...]
    h = jnp.where(colmask, jax.nn.sigmoid(hlat), 0.0)
    h_ref[...] = h

    def loss_blk(i, acc):
        u0 = i * _UB
        _, praw, tcol = block_fwd(u0)
        return acc + jnp.sum((praw - tcol) ** 2)

    odd_sum = jax.lax.fori_loop(0, _NBLK, loss_blk, jnp.float32(0.0),
                                unroll=False)
    loss_odd = odd_sum / (4.0 * _NM)

    tau = tau_ref[...]
    tau_col = jnp.transpose(tau)[: _NM, :]
    r = jnp.maximum(tau_col - tau + 0.1, 0.0)
    loss_order = jnp.sum(h * (r * r)) * _INV_MEAN
    out_ref[0, 0] = loss_odd + loss_order


def kernel(A, tau_init, G_latent_init):
    aw2 = 2.0 * A[: _NM, :]
    hlat0 = jnp.pad(G_latent_init, ((0, 0), (_PAD, 0)))
    tau0 = tau_init.reshape(1, _N)

    out = pl.pallas_call(
        _body,
        out_shape=jax.ShapeDtypeStruct((1, 1), jnp.float32),
        out_specs=pl.BlockSpec(memory_space=pltpu.SMEM),
        scratch_shapes=[
            pltpu.VMEM((_NM, _N), jnp.float32),
            pltpu.VMEM((_NM, _N), jnp.float32),
            pltpu.VMEM((_NM, _N), jnp.float32),
            pltpu.VMEM((_NM, _N), jnp.float32),
            pltpu.VMEM((_NM, _N), jnp.float32),
            pltpu.VMEM((1, _N), jnp.float32),
            pltpu.VMEM((1, _N), jnp.float32),
            pltpu.VMEM((1, _N), jnp.float32),
        ],
    )(aw2, hlat0, tau0)
    return out[0, 0]
